# baseline (speedup 1.0000x reference)
import functools

import jax
import jax.numpy as jnp
from jax import lax
from jax.experimental import pallas as pl
from jax.experimental.pallas import tpu as pltpu
from jax.experimental.pallas import tpu_sc as plsc

NUM_LEVELS = 16
LEVEL_DIM = 2
TABLE_SIZE = 1 << 19
RESOLUTION = 512
H1 = 2654435761
H2 = 805459861

_RES = [16 * (2 ** l) for l in range(NUM_LEVELS)]
_DENSE = [(r + 2) ** 3 <= TABLE_SIZE for r in _RES]
LOW_LEVELS = 5
N_SLOTS = LOW_LEVELS * 8 + (NUM_LEVELS - LOW_LEVELS)
NW = 32
_MASK = TABLE_SIZE - 1


def _corner_coords(xyz_ref, b):
    cs, c0s = [], []
    for a in range(3):
        x = xyz_ref[:, a]
        coord = (x + b) / (2.0 * b) * float(RESOLUTION)
        c0 = jnp.clip(jnp.floor(coord), 0.0, float(RESOLUTION - 1)).astype(jnp.int32)
        cs.append(coord)
        c0s.append(c0)
    return cs, c0s


def _hash3(gx, gy, gz):
    h = (gx.astype(jnp.uint32)
         ^ (gy.astype(jnp.uint32) * jnp.uint32(H1))
         ^ (gz.astype(jnp.uint32) * jnp.uint32(H2)))
    return (h & jnp.uint32(_MASK)).astype(jnp.int32)


def _prep_kernel(xyz_ref, b_ref, idx_ref):
    b = b_ref[0, 0]
    _, c0s = _corner_coords(xyz_ref, b)

    def body(k, carry):
        dk = ((k >> 2) & 1, (k >> 1) & 1, k & 1)
        cx = c0s[0] + dk[0]
        cy = c0s[1] + dk[1]
        cz = c0s[2] + dk[2]
        for l in range(LOW_LEVELS):
            sh = LOW_LEVELS - l
            p0x, p0y, p0z = cx >> sh, cy >> sh, cz >> sh
            for j in range(8):
                ax, ay, az = (j >> 2) & 1, (j >> 1) & 1, j & 1
                gx, gy, gz = p0x + ax, p0y + ay, p0z + az
                if _DENSE[l]:
                    stride = _RES[l] + 2
                    row = gx + stride * gy + (stride * stride) * gz
                else:
                    row = _hash3(gx, gy, gz)
                idx_ref[8 * l + j, k, :] = row + (l << 19)
        for l in range(LOW_LEVELS, NUM_LEVELS):
            shl = l - LOW_LEVELS
            row = _hash3(cx << shl, cy << shl, cz << shl)
            idx_ref[40 + shl, k, :] = row + (l << 19)
        return carry

    lax.fori_loop(0, 8, body, 0)


def _prep_call(xyz, bf):
    n = xyz.shape[0]
    bp = min(512, n)
    return pl.pallas_call(
        _prep_kernel,
        grid=(n // bp,),
        in_specs=[pl.BlockSpec((bp, 3), lambda i: (i, 0)),
                  pl.BlockSpec((1, 1), lambda i: (0, 0))],
        out_specs=pl.BlockSpec((N_SLOTS, 8, bp), lambda i: (0, 0, i)),
        out_shape=jax.ShapeDtypeStruct((N_SLOTS, 8, n), jnp.int32),
    )(xyz, bf)


def _sc_gather(tab0, tab1, idx3):
    p_tot, g128, _ = idx3.shape
    cw = g128 // NW
    mesh = plsc.VectorSubcoreMesh(core_axis_name="c", subcore_axis_name="s")

    @functools.partial(
        pl.kernel,
        out_type=jax.ShapeDtypeStruct((2, p_tot, g128, 128), jnp.float32),
        mesh=mesh,
        scratch_types=[
            pltpu.VMEM((cw, 128), jnp.int32),
            pltpu.VMEM((cw, 128), jnp.float32),
            pltpu.VMEM((cw, 128), jnp.float32),
            pltpu.SemaphoreType.DMA,
            pltpu.SemaphoreType.DMA,
        ],
    )
    def k(t0_hbm, t1_hbm, idx_hbm, out_hbm, idx_v, g0_v, g1_v, gsem, csem):
        wid = lax.axis_index("s") * 2 + lax.axis_index("c")
        base = wid * cw

        @pl.loop(0, p_tot)
        def _(p):
            pltpu.async_copy(idx_hbm.at[p, pl.ds(base, cw)], idx_v, csem).wait()

            @pl.loop(0, cw, step=8)
            def _(g0):
                for t in range(8):
                    pltpu.async_copy(t0_hbm.at[idx_v.at[g0 + t]],
                                     g0_v.at[g0 + t], gsem)
                    pltpu.async_copy(t1_hbm.at[idx_v.at[g0 + t]],
                                     g1_v.at[g0 + t], gsem)
                for t in range(8):
                    pltpu.make_async_copy(t0_hbm.at[idx_v.at[g0 + t]],
                                          g0_v.at[g0 + t], gsem).wait()
                    pltpu.make_async_copy(t1_hbm.at[idx_v.at[g0 + t]],
                                          g1_v.at[g0 + t], gsem).wait()

            pltpu.async_copy(g0_v, out_hbm.at[0, p, pl.ds(base, cw)], csem).wait()
            pltpu.async_copy(g1_v, out_hbm.at[1, p, pl.ds(base, cw)], csem).wait()

    return k(tab0, tab1, idx3)


def _mlp_kernel(xyz_ref, b_ref, gath_ref, w0t_ref, w1t_ref, w2t_ref, out_ref):
    b = b_ref[0, 0]
    cs, c0s = _corner_coords(xyz_ref, b)
    uvw = [cs[a] - c0s[a].astype(jnp.float32) for a in range(3)]
    w0t = w0t_ref[...]
    w1t = w1t_ref[...]
    w2t = w2t_ref[...]
    def body(k, acc):
        dk = ((k >> 2) & 1, (k >> 1) & 1, k & 1)
        dkf = [d.astype(jnp.float32) for d in dk]
        cf = [(c0s[a] + dk[a]).astype(jnp.float32) for a in range(3)]
        cols = []
        for l in range(LOW_LEVELS):
            s_l = float(2.0 ** (l - LOW_LEVELS))
            ts = []
            for a in range(3):
                pos = cf[a] * s_l
                ts.append(pos - jnp.floor(pos))
            f0 = jnp.zeros_like(ts[0])
            f1 = jnp.zeros_like(ts[0])
            for j in range(8):
                ax, ay, az = (j >> 2) & 1, (j >> 1) & 1, j & 1
                wx = ts[0] if ax else 1.0 - ts[0]
                wy = ts[1] if ay else 1.0 - ts[1]
                wz = ts[2] if az else 1.0 - ts[2]
                wj = (wx * wy) * wz
                f0 = f0 + wj * gath_ref[0, 8 * l + j, k]
                f1 = f1 + wj * gath_ref[1, 8 * l + j, k]
            cols.append(f0)
            cols.append(f1)
        for l in range(LOW_LEVELS, NUM_LEVELS):
            s = 40 + (l - LOW_LEVELS)
            cols.append(gath_ref[0, s, k])
            cols.append(gath_ref[1, s, k])
        xt = jnp.stack(cols, axis=0)
        h = jnp.maximum(jnp.dot(w0t, xt, preferred_element_type=jnp.float32,
                                precision=lax.Precision.HIGHEST), 0.0)
        h = jnp.maximum(jnp.dot(w1t, h, preferred_element_type=jnp.float32,
                                precision=lax.Precision.HIGHEST), 0.0)
        o = jnp.dot(w2t, h, preferred_element_type=jnp.float32,
                    precision=lax.Precision.HIGHEST)
        tx = dkf[0] * uvw[0] + (1.0 - dkf[0]) * (1.0 - uvw[0])
        ty = dkf[1] * uvw[1] + (1.0 - dkf[1]) * (1.0 - uvw[1])
        tz = dkf[2] * uvw[2] + (1.0 - dkf[2]) * (1.0 - uvw[2])
        wk = (tz * ty) * tx
        return acc + wk[None, :] * o

    out_ref[...] = lax.fori_loop(0, 8, body, jnp.zeros(out_ref.shape, jnp.float32))


def _mlp_call(xyz, bf, gath4, w0t, w1t, w2t):
    n = xyz.shape[0]
    bm = min(512, n)
    in_dim = NUM_LEVELS * LEVEL_DIM
    hidden = w0t.shape[0]
    out_dim = w2t.shape[0]
    return pl.pallas_call(
        _mlp_kernel,
        grid=(n // bm,),
        in_specs=[
            pl.BlockSpec((bm, 3), lambda i: (i, 0)),
            pl.BlockSpec((1, 1), lambda i: (0, 0)),
            pl.BlockSpec((2, N_SLOTS, 8, bm), lambda i: (0, 0, 0, i)),
            pl.BlockSpec((hidden, in_dim), lambda i: (0, 0)),
            pl.BlockSpec((hidden, hidden), lambda i: (0, 0)),
            pl.BlockSpec((out_dim, hidden), lambda i: (0, 0)),
        ],
        out_specs=pl.BlockSpec((out_dim, bm), lambda i: (0, i)),
        out_shape=jax.ShapeDtypeStruct((out_dim, n), jnp.float32),
    )(xyz, bf, gath4, w0t, w1t, w2t)


def kernel(xyz, bound, tables, W0, W1, W2):
    n = xyz.shape[0]
    bf = jnp.asarray(bound, jnp.float32).reshape(1, 1)
    tabs = tables.reshape(NUM_LEVELS * TABLE_SIZE, LEVEL_DIM)
    tab0 = tabs[:, 0]
    tab1 = tabs[:, 1]
    idx = _prep_call(xyz, bf)
    idx3 = idx.reshape(N_SLOTS * 8, n // 128, 128)
    gath = _sc_gather(tab0, tab1, idx3)
    gath4 = gath.reshape(2, N_SLOTS, 8, n)
    out_t = _mlp_call(xyz, bf, gath4, W0.T, W1.T, W2.T)
    return out_t.T


# speedup vs baseline: 1.0091x; 1.0091x over previous
import functools

import jax
import jax.numpy as jnp
from jax import lax
from jax.experimental import pallas as pl
from jax.experimental.pallas import tpu as pltpu
from jax.experimental.pallas import tpu_sc as plsc

NUM_LEVELS = 16
LEVEL_DIM = 2
TABLE_SIZE = 1 << 19
RESOLUTION = 512
H1 = 2654435761
H2 = 805459861

_RES = [16 * (2 ** l) for l in range(NUM_LEVELS)]
_DENSE = [(r + 2) ** 3 <= TABLE_SIZE for r in _RES]
LOW_LEVELS = 5
N_SLOTS = LOW_LEVELS * 8 + (NUM_LEVELS - LOW_LEVELS)
NW = 32
_MASK = TABLE_SIZE - 1


def _corner_coords(xyz_ref, b):
    cs, c0s = [], []
    for a in range(3):
        x = xyz_ref[:, a]
        coord = (x + b) / (2.0 * b) * float(RESOLUTION)
        c0 = jnp.clip(jnp.floor(coord), 0.0, float(RESOLUTION - 1)).astype(jnp.int32)
        cs.append(coord)
        c0s.append(c0)
    return cs, c0s


def _hash3(gx, gy, gz):
    h = (gx.astype(jnp.uint32)
         ^ (gy.astype(jnp.uint32) * jnp.uint32(H1))
         ^ (gz.astype(jnp.uint32) * jnp.uint32(H2)))
    return (h & jnp.uint32(_MASK)).astype(jnp.int32)


def _prep_kernel(xyz_ref, b_ref, idx_ref):
    b = b_ref[0, 0]
    _, c0s = _corner_coords(xyz_ref, b)

    def body(k, carry):
        dk = ((k >> 2) & 1, (k >> 1) & 1, k & 1)
        cx = c0s[0] + dk[0]
        cy = c0s[1] + dk[1]
        cz = c0s[2] + dk[2]
        for l in range(LOW_LEVELS):
            sh = LOW_LEVELS - l
            p0x, p0y, p0z = cx >> sh, cy >> sh, cz >> sh
            for j in range(8):
                ax, ay, az = (j >> 2) & 1, (j >> 1) & 1, j & 1
                gx, gy, gz = p0x + ax, p0y + ay, p0z + az
                if _DENSE[l]:
                    stride = _RES[l] + 2
                    row = gx + stride * gy + (stride * stride) * gz
                else:
                    row = _hash3(gx, gy, gz)
                idx_ref[8 * l + j, k, :] = row + (l << 19)
        for l in range(LOW_LEVELS, NUM_LEVELS):
            shl = l - LOW_LEVELS
            row = _hash3(cx << shl, cy << shl, cz << shl)
            idx_ref[40 + shl, k, :] = row + (l << 19)
        return carry

    lax.fori_loop(0, 8, body, 0)


def _prep_call(xyz, bf):
    n = xyz.shape[0]
    bp = min(512, n)
    return pl.pallas_call(
        _prep_kernel,
        grid=(n // bp,),
        in_specs=[pl.BlockSpec((bp, 3), lambda i: (i, 0)),
                  pl.BlockSpec((1, 1), lambda i: (0, 0))],
        out_specs=pl.BlockSpec((N_SLOTS, 8, bp), lambda i: (0, 0, i)),
        out_shape=jax.ShapeDtypeStruct((N_SLOTS, 8, n), jnp.int32),
    )(xyz, bf)


def _sc_gather(tab0, tab1, idx3):
    p_tot, g128, _ = idx3.shape
    cw = g128 // NW
    mesh = plsc.VectorSubcoreMesh(core_axis_name="c", subcore_axis_name="s")

    @functools.partial(
        pl.kernel,
        out_type=jax.ShapeDtypeStruct((2, p_tot, g128, 128), jnp.float32),
        mesh=mesh,
        scratch_types=[
            pltpu.VMEM((cw, 128), jnp.int32),
            pltpu.VMEM((cw, 128), jnp.float32),
            pltpu.VMEM((cw, 128), jnp.float32),
            pltpu.SemaphoreType.DMA,
            pltpu.SemaphoreType.DMA,
        ],
    )
    def k(t0_hbm, t1_hbm, idx_hbm, out_hbm, idx_v, g0_v, g1_v, gsem, csem):
        wid = lax.axis_index("s") * 2 + lax.axis_index("c")
        base = wid * cw

        @pl.loop(0, p_tot)
        def _(p):
            pltpu.async_copy(idx_hbm.at[p, pl.ds(base, cw)], idx_v, csem).wait()

            @pl.loop(0, cw, step=4)
            def _(g0):
                for t in range(4):
                    pltpu.async_copy(t0_hbm.at[idx_v.at[g0 + t]],
                                     g0_v.at[g0 + t], gsem)
                    pltpu.async_copy(t1_hbm.at[idx_v.at[g0 + t]],
                                     g1_v.at[g0 + t], gsem)

            @pl.loop(0, cw, step=4)
            def _(g0):
                for t in range(4):
                    pltpu.make_async_copy(t0_hbm.at[idx_v.at[g0 + t]],
                                          g0_v.at[g0 + t], gsem).wait()
                    pltpu.make_async_copy(t1_hbm.at[idx_v.at[g0 + t]],
                                          g1_v.at[g0 + t], gsem).wait()

            pltpu.async_copy(g0_v, out_hbm.at[0, p, pl.ds(base, cw)], csem).wait()
            pltpu.async_copy(g1_v, out_hbm.at[1, p, pl.ds(base, cw)], csem).wait()

    return k(tab0, tab1, idx3)


def _mlp_kernel(xyz_ref, b_ref, gath_ref, w0t_ref, w1t_ref, w2t_ref, out_ref):
    b = b_ref[0, 0]
    cs, c0s = _corner_coords(xyz_ref, b)
    uvw = [cs[a] - c0s[a].astype(jnp.float32) for a in range(3)]
    w0t = w0t_ref[...]
    w1t = w1t_ref[...]
    w2t = w2t_ref[...]
    def body(k, acc):
        dk = ((k >> 2) & 1, (k >> 1) & 1, k & 1)
        dkf = [d.astype(jnp.float32) for d in dk]
        cf = [(c0s[a] + dk[a]).astype(jnp.float32) for a in range(3)]
        cols = []
        for l in range(LOW_LEVELS):
            s_l = float(2.0 ** (l - LOW_LEVELS))
            ts = []
            for a in range(3):
                pos = cf[a] * s_l
                ts.append(pos - jnp.floor(pos))
            f0 = jnp.zeros_like(ts[0])
            f1 = jnp.zeros_like(ts[0])
            for j in range(8):
                ax, ay, az = (j >> 2) & 1, (j >> 1) & 1, j & 1
                wx = ts[0] if ax else 1.0 - ts[0]
                wy = ts[1] if ay else 1.0 - ts[1]
                wz = ts[2] if az else 1.0 - ts[2]
                wj = (wx * wy) * wz
                f0 = f0 + wj * gath_ref[0, 8 * l + j, k]
                f1 = f1 + wj * gath_ref[1, 8 * l + j, k]
            cols.append(f0)
            cols.append(f1)
        for l in range(LOW_LEVELS, NUM_LEVELS):
            s = 40 + (l - LOW_LEVELS)
            cols.append(gath_ref[0, s, k])
            cols.append(gath_ref[1, s, k])
        xt = jnp.stack(cols, axis=0)
        h = jnp.maximum(jnp.dot(w0t, xt, preferred_element_type=jnp.float32,
                                precision=lax.Precision.HIGHEST), 0.0)
        h = jnp.maximum(jnp.dot(w1t, h, preferred_element_type=jnp.float32,
                                precision=lax.Precision.HIGHEST), 0.0)
        o = jnp.dot(w2t, h, preferred_element_type=jnp.float32,
                    precision=lax.Precision.HIGHEST)
        tx = dkf[0] * uvw[0] + (1.0 - dkf[0]) * (1.0 - uvw[0])
        ty = dkf[1] * uvw[1] + (1.0 - dkf[1]) * (1.0 - uvw[1])
        tz = dkf[2] * uvw[2] + (1.0 - dkf[2]) * (1.0 - uvw[2])
        wk = (tz * ty) * tx
        return acc + wk[None, :] * o

    out_ref[...] = lax.fori_loop(0, 8, body, jnp.zeros(out_ref.shape, jnp.float32))


def _mlp_call(xyz, bf, gath4, w0t, w1t, w2t):
    n = xyz.shape[0]
    bm = min(512, n)
    in_dim = NUM_LEVELS * LEVEL_DIM
    hidden = w0t.shape[0]
    out_dim = w2t.shape[0]
    return pl.pallas_call(
        _mlp_kernel,
        grid=(n // bm,),
        in_specs=[
            pl.BlockSpec((bm, 3), lambda i: (i, 0)),
            pl.BlockSpec((1, 1), lambda i: (0, 0)),
            pl.BlockSpec((2, N_SLOTS, 8, bm), lambda i: (0, 0, 0, i)),
            pl.BlockSpec((hidden, in_dim), lambda i: (0, 0)),
            pl.BlockSpec((hidden, hidden), lambda i: (0, 0)),
            pl.BlockSpec((out_dim, hidden), lambda i: (0, 0)),
        ],
        out_specs=pl.BlockSpec((out_dim, bm), lambda i: (0, i)),
        out_shape=jax.ShapeDtypeStruct((out_dim, n), jnp.float32),
    )(xyz, bf, gath4, w0t, w1t, w2t)


def kernel(xyz, bound, tables, W0, W1, W2):
    n = xyz.shape[0]
    bf = jnp.asarray(bound, jnp.float32).reshape(1, 1)
    tabs = tables.reshape(NUM_LEVELS * TABLE_SIZE, LEVEL_DIM)
    tab0 = tabs[:, 0]
    tab1 = tabs[:, 1]
    idx = _prep_call(xyz, bf)
    idx3 = idx.reshape(N_SLOTS * 8, n // 128, 128)
    gath = _sc_gather(tab0, tab1, idx3)
    gath4 = gath.reshape(2, N_SLOTS, 8, n)
    out_t = _mlp_call(xyz, bf, gath4, W0.T, W1.T, W2.T)
    return out_t.T
